# baseline (speedup 1.0000x reference)
import jax
import jax.numpy as jnp
from jax import lax
from jax.experimental import pallas as pl
from jax.experimental.pallas import tpu as pltpu

ALPHA = 0.7
TAU_SENT = 1.0
_W = 128


def _loss_body(ts_ref, x_ref, t_ref, m_ref, ml_ref, tot_ref, acc_ref):
    i = pl.program_id(0)
    nb = pl.num_programs(0)
    t = t_ref[0]
    m = m_ref[0]
    _, S, V = x_ref.shape

    x = x_ref[0]
    maxv = jnp.max(x, axis=1, keepdims=True)
    idx = jnp.argmax(x, axis=1).reshape(S, 1)
    match = (idx == t).astype(jnp.float32)

    lane2 = lax.broadcasted_iota(jnp.int32, (1, _W), 1)
    tvs = []
    for s in range(S):
        start = pl.multiple_of(ts_ref[i, s], _W)
        vs = x_ref[0, pl.ds(s, 1), pl.ds(start, _W)]
        tmod = ts_ref[i + nb, s]
        tvs.append(jnp.sum(jnp.where(lane2 == tmod, vs, 0.0), axis=1,
                           keepdims=True))
    tval = jnp.concatenate(tvs, axis=0)

    sent = jnp.exp(jnp.sum(match, axis=0, keepdims=True)
                   * (1.0 / (S * TAU_SENT)))
    mlp = jnp.sum(tval * m, axis=0, keepdims=True)
    msp = jnp.sum(m, axis=0, keepdims=True)
    outp = sent * jnp.sum(maxv * m, axis=0, keepdims=True)
    denp = sent * msp

    @pl.when(i == 0)
    def _init():
        acc_ref[...] = jnp.zeros_like(acc_ref)

    acc_ref[0:1, 0:1] += mlp
    acc_ref[1:2, 0:1] += msp
    acc_ref[2:3, 0:1] += outp
    acc_ref[3:4, 0:1] += denp

    @pl.when(i == nb - 1)
    def _finish():
        ml = -acc_ref[0:1, 0:1] / acc_ref[1:2, 0:1]
        out = -acc_ref[2:3, 0:1] / acc_ref[3:4, 0:1]
        ml_ref[...] = ml
        tot_ref[...] = ALPHA * out + (1.0 - ALPHA) * ml


def kernel(input, target, mask):
    B, S, V = input.shape
    t32 = target.astype(jnp.int32)
    t3 = t32.reshape(B, S, 1)
    m3 = mask.astype(jnp.float32).reshape(B, S, 1)
    tpre = jnp.concatenate([(t32 // _W) * _W, t32 % _W], axis=0)

    ml, tot = pl.pallas_call(
        _loss_body,
        grid=(B,),
        in_specs=[
            pl.BlockSpec(memory_space=pltpu.SMEM),
            pl.BlockSpec((1, S, V), lambda i: (i, 0, 0)),
            pl.BlockSpec((1, S, 1), lambda i: (i, 0, 0)),
            pl.BlockSpec((1, S, 1), lambda i: (i, 0, 0)),
        ],
        out_specs=[
            pl.BlockSpec((1, 1), lambda i: (0, 0)),
            pl.BlockSpec((1, 1), lambda i: (0, 0)),
        ],
        out_shape=[
            jax.ShapeDtypeStruct((1, 1), jnp.float32),
            jax.ShapeDtypeStruct((1, 1), jnp.float32),
        ],
        scratch_shapes=[pltpu.VMEM((8, 128), jnp.float32)],
    )(tpre, input, t3, m3)
    return (ml.reshape(()), tot.reshape(()))


# speedup vs baseline: 1.2655x; 1.2655x over previous
import jax
import jax.numpy as jnp
from jax import lax
from jax.experimental import pallas as pl
from jax.experimental.pallas import tpu as pltpu

ALPHA = 0.7
TAU_SENT = 1.0
_W = 128
_BB = 2


def _loss_body(ts_ref, x_ref, t_ref, m_ref, ml_ref, tot_ref, acc_ref):
    i = pl.program_id(0)
    nb = pl.num_programs(0)
    t = t_ref[...]
    m = m_ref[...]
    BB, S, V = x_ref.shape

    x = x_ref[...]
    iota = lax.broadcasted_iota(jnp.int32, (BB, S, V), 2).astype(jnp.float32)
    tf = t.astype(jnp.float32)
    maxv = jnp.max(x, axis=2, keepdims=True)
    idxf = jnp.min(jnp.where(x == maxv, iota, 3.4e38), axis=2,
                   keepdims=True)
    match = (idxf == tf).astype(jnp.float32)

    lane2 = lax.broadcasted_iota(jnp.int32, (1, _W), 1)
    tvs = []
    for b in range(BB):
        for s in range(S):
            start = pl.multiple_of(ts_ref[i * _BB + b, s], _W)
            vs = x_ref[b, pl.ds(s, 1), pl.ds(start, _W)]
            tmod = ts_ref[i * _BB + b + nb * _BB, s]
            tvs.append(jnp.sum(jnp.where(lane2 == tmod, vs, 0.0),
                               axis=1, keepdims=True))
    tval = jnp.concatenate(tvs, axis=0).reshape(BB, S, 1)

    sent = jnp.exp(jnp.sum(match, axis=1, keepdims=True)
                   * (1.0 / (S * TAU_SENT)))
    mlp = jnp.sum(tval * m, axis=(0, 1), keepdims=True)
    msp = jnp.sum(m, axis=(0, 1), keepdims=True)
    outp = jnp.sum(sent * jnp.sum(maxv * m, axis=1, keepdims=True),
                   axis=0, keepdims=True)
    denp = jnp.sum(sent * jnp.sum(m, axis=1, keepdims=True),
                   axis=0, keepdims=True)

    @pl.when(i == 0)
    def _init():
        acc_ref[...] = jnp.zeros_like(acc_ref)

    acc_ref[0:1, 0:1] += mlp[0]
    acc_ref[1:2, 0:1] += msp[0]
    acc_ref[2:3, 0:1] += outp[0]
    acc_ref[3:4, 0:1] += denp[0]

    @pl.when(i == nb - 1)
    def _finish():
        ml = -acc_ref[0:1, 0:1] / acc_ref[1:2, 0:1]
        out = -acc_ref[2:3, 0:1] / acc_ref[3:4, 0:1]
        ml_ref[...] = ml
        tot_ref[...] = ALPHA * out + (1.0 - ALPHA) * ml


def kernel(input, target, mask):
    B, S, V = input.shape
    t32 = target.astype(jnp.int32)
    t3 = t32.reshape(B, S, 1)
    m3 = mask.astype(jnp.float32).reshape(B, S, 1)
    tpre = jnp.concatenate([(t32 // _W) * _W, t32 % _W], axis=0)

    ml, tot = pl.pallas_call(
        _loss_body,
        grid=(B // _BB,),
        in_specs=[
            pl.BlockSpec(memory_space=pltpu.SMEM),
            pl.BlockSpec((_BB, S, V), lambda i: (i, 0, 0)),
            pl.BlockSpec((_BB, S, 1), lambda i: (i, 0, 0)),
            pl.BlockSpec((_BB, S, 1), lambda i: (i, 0, 0)),
        ],
        out_specs=[
            pl.BlockSpec((1, 1), lambda i: (0, 0)),
            pl.BlockSpec((1, 1), lambda i: (0, 0)),
        ],
        out_shape=[
            jax.ShapeDtypeStruct((1, 1), jnp.float32),
            jax.ShapeDtypeStruct((1, 1), jnp.float32),
        ],
        scratch_shapes=[pltpu.VMEM((8, 128), jnp.float32)],
    )(tpre, input, t3, m3)
    return (ml.reshape(()), tot.reshape(()))


# speedup vs baseline: 1.3376x; 1.0570x over previous
import jax
import jax.numpy as jnp
from jax import lax
from jax.experimental import pallas as pl
from jax.experimental.pallas import tpu as pltpu

ALPHA = 0.7
TAU_SENT = 1.0
_W = 128
_BB = 4


def _loss_body(ts_ref, x_ref, t_ref, m_ref, ml_ref, tot_ref, acc_ref):
    i = pl.program_id(0)
    nb = pl.num_programs(0)
    t = t_ref[...]
    m = m_ref[...]
    BB, S, V = x_ref.shape

    x = x_ref[...]
    iota = lax.broadcasted_iota(jnp.int32, (BB, S, V), 2).astype(jnp.float32)
    tf = t.astype(jnp.float32)
    maxv = jnp.max(x, axis=2, keepdims=True)
    idxf = jnp.min(jnp.where(x == maxv, iota, 3.4e38), axis=2,
                   keepdims=True)
    match = (idxf == tf).astype(jnp.float32)

    lane2 = lax.broadcasted_iota(jnp.int32, (1, _W), 1)
    tvs = []
    for b in range(BB):
        for s in range(S):
            start = pl.multiple_of(ts_ref[i * _BB + b, s], _W)
            vs = x_ref[b, pl.ds(s, 1), pl.ds(start, _W)]
            tmod = ts_ref[i * _BB + b + nb * _BB, s]
            tvs.append(jnp.sum(jnp.where(lane2 == tmod, vs, 0.0),
                               axis=1, keepdims=True))
    tval = jnp.concatenate(tvs, axis=0).reshape(BB, S, 1)

    sent = jnp.exp(jnp.sum(match, axis=1, keepdims=True)
                   * (1.0 / (S * TAU_SENT)))
    mlp = jnp.sum(tval * m, axis=(0, 1), keepdims=True)
    msp = jnp.sum(m, axis=(0, 1), keepdims=True)
    outp = jnp.sum(sent * jnp.sum(maxv * m, axis=1, keepdims=True),
                   axis=0, keepdims=True)
    denp = jnp.sum(sent * jnp.sum(m, axis=1, keepdims=True),
                   axis=0, keepdims=True)

    @pl.when(i == 0)
    def _init():
        acc_ref[...] = jnp.zeros_like(acc_ref)

    acc_ref[0:1, 0:1] += mlp[0]
    acc_ref[1:2, 0:1] += msp[0]
    acc_ref[2:3, 0:1] += outp[0]
    acc_ref[3:4, 0:1] += denp[0]

    @pl.when(i == nb - 1)
    def _finish():
        ml = -acc_ref[0:1, 0:1] / acc_ref[1:2, 0:1]
        out = -acc_ref[2:3, 0:1] / acc_ref[3:4, 0:1]
        ml_ref[...] = ml
        tot_ref[...] = ALPHA * out + (1.0 - ALPHA) * ml


def kernel(input, target, mask):
    B, S, V = input.shape
    t32 = target.astype(jnp.int32)
    t3 = t32.reshape(B, S, 1)
    m3 = mask.astype(jnp.float32).reshape(B, S, 1)
    tpre = jnp.concatenate([(t32 // _W) * _W, t32 % _W], axis=0)

    ml, tot = pl.pallas_call(
        _loss_body,
        grid=(B // _BB,),
        in_specs=[
            pl.BlockSpec(memory_space=pltpu.SMEM),
            pl.BlockSpec((_BB, S, V), lambda i: (i, 0, 0)),
            pl.BlockSpec((_BB, S, 1), lambda i: (i, 0, 0)),
            pl.BlockSpec((_BB, S, 1), lambda i: (i, 0, 0)),
        ],
        out_specs=[
            pl.BlockSpec((1, 1), lambda i: (0, 0)),
            pl.BlockSpec((1, 1), lambda i: (0, 0)),
        ],
        out_shape=[
            jax.ShapeDtypeStruct((1, 1), jnp.float32),
            jax.ShapeDtypeStruct((1, 1), jnp.float32),
        ],
        scratch_shapes=[pltpu.VMEM((8, 128), jnp.float32)],
    )(tpre, input, t3, m3)
    return (ml.reshape(()), tot.reshape(()))
